# baseline (speedup 1.0000x reference)
import functools

import jax
import jax.numpy as jnp
from jax import lax
from jax.experimental import pallas as pl
from jax.experimental.pallas import tpu as pltpu
from jax.experimental.pallas import tpu_sc as plsc

NC, NS = 2, 16
NW = NC * NS
B, L, D = 4096, 200, 64
EN = 5
SB = 4
K = 25
G = L // K
BPW = B // NW
S = BPW // SB
IPG = K * SB


def _arrange(text):
    a = text.astype(jnp.int32).reshape(NW, S, SB, G, K)
    a = a.transpose(0, 1, 3, 4, 2)
    return a.reshape(NW, S * G, IPG)


def _sc_body(text_ref, table_ref, out_ref, idx_v, rows_v, out_v, sem):
    w = lax.axis_index("s") * NC + lax.axis_index("c")
    pltpu.sync_copy(text_ref.at[w], idx_v)

    def sub_block(s, _):
        accs = tuple(jnp.zeros((16,), jnp.float32) for _ in range(SB * 4))
        for g in range(G):
            buf = g % 2
            pltpu.async_copy(
                table_ref.at[idx_v.at[s * G + g]], rows_v.at[buf], sem
            ).wait()

            def kbody(k, acc):
                new = []
                for t in range(SB):
                    for c in range(4):
                        v = rows_v[buf, k * SB + t, pl.ds(c * 16, 16)]
                        new.append(acc[t * 4 + c] + v)
                return tuple(new)

            accs = lax.fori_loop(0, K, kbody, accs)
        for t in range(SB):
            for c in range(4):
                out_v[s * SB + t, pl.ds(c * 16, 16)] = accs[t * 4 + c]
        return 0

    lax.fori_loop(0, S, sub_block, 0)
    pltpu.sync_copy(out_v, out_ref.at[pl.ds(w * BPW, BPW)])


def _sc_sums(arr, table):
    mesh = plsc.VectorSubcoreMesh(core_axis_name="c", subcore_axis_name="s")
    return pl.kernel(
        _sc_body,
        out_type=jax.ShapeDtypeStruct((B, D), jnp.float32),
        mesh=mesh,
        scratch_types=[
            pltpu.VMEM((S * G, IPG), jnp.int32),
            pltpu.VMEM((2, IPG, D), jnp.float32),
            pltpu.VMEM((BPW, D), jnp.float32),
            pltpu.SemaphoreType.DMA,
        ],
        compiler_params=pltpu.CompilerParams(use_tc_tiling_on_sc=False),
    )(arr, table)


def _tc_tail_body(s_ref, w_ref, b_ref, o_ref):
    x = s_ref[...] * (1.0 / L)
    logits = lax.dot_general(
        x, w_ref[...], (((1,), (1,)), ((), ())),
        preferred_element_type=jnp.float32,
    ) + b_ref[...]
    m = jnp.max(logits, axis=1, keepdims=True)
    sh = logits - m
    o_ref[...] = sh - jnp.log(jnp.sum(jnp.exp(sh), axis=1, keepdims=True))


def _tc_tail(sums, W, b):
    return pl.pallas_call(
        _tc_tail_body,
        out_shape=jax.ShapeDtypeStruct((B, EN), jnp.float32),
    )(sums, W, b.reshape(1, EN))


def kernel(text, table, W, b):
    sums = _sc_sums(_arrange(text), table)
    return _tc_tail(sums, W, b)


# speedup vs baseline: 1.0897x; 1.0897x over previous
import jax
import jax.numpy as jnp
from jax import lax
from jax.experimental import pallas as pl
from jax.experimental.pallas import tpu as pltpu
from jax.experimental.pallas import tpu_sc as plsc

NC, NS = 2, 16
NW = NC * NS
B, L, D = 4096, 200, 64
V = 1000000
EN = 5
PD = 16
IPG = 100
GPW = 2 * (B // NW)
BPW = B // NW
PC = 8192


def _project_body(w_ref, t_ref, o_ref):
    o_ref[...] = lax.dot_general(
        w_ref[...], t_ref[...], (((1,), (0,)), ((), ())),
        preferred_element_type=jnp.float32,
    )


def _project(tableT, W16s):
    nb = (V + PC - 1) // PC
    return pl.pallas_call(
        _project_body,
        grid=(nb,),
        in_specs=[
            pl.BlockSpec((PD, D), lambda i: (0, 0)),
            pl.BlockSpec((D, PC), lambda i: (0, i)),
        ],
        out_specs=pl.BlockSpec((PD, PC), lambda i: (0, i)),
        out_shape=jax.ShapeDtypeStruct((PD, V), jnp.float32),
    )(W16s, tableT)


def _sc_body(idx_hbm, p_hbm, out_hbm, idx_v, rows_v, out_v, sem0, sem1):
    w = lax.axis_index("s") * NC + lax.axis_index("c")
    pltpu.sync_copy(idx_hbm.at[w], idx_v)
    sems = (sem0, sem1)

    pltpu.async_copy(p_hbm.at[idx_v.at[0]], rows_v.at[0], sem0)

    def row_body(r, _):
        acc = jnp.zeros((PD,), jnp.float32)
        for h in range(2):
            g = r * 2 + h
            nxt = jnp.minimum(g + 1, GPW - 1)
            pltpu.async_copy(
                p_hbm.at[idx_v.at[nxt]], rows_v.at[1 - h], sems[1 - h]
            )
            pltpu.make_async_copy(
                p_hbm.at[idx_v.at[g]], rows_v.at[h], sems[h]
            ).wait()

            def kbody(k, a):
                r0 = rows_v[h, k * 4 + 0, :]
                r1 = rows_v[h, k * 4 + 1, :]
                r2 = rows_v[h, k * 4 + 2, :]
                r3 = rows_v[h, k * 4 + 3, :]
                return a + ((r0 + r1) + (r2 + r3))

            acc = lax.fori_loop(0, IPG // 4, kbody, acc)
        out_v[r, :] = acc
        return 0

    lax.fori_loop(0, BPW, row_body, 0)
    pltpu.make_async_copy(
        p_hbm.at[idx_v.at[GPW - 1]], rows_v.at[0], sem0
    ).wait()
    pltpu.sync_copy(out_v, out_hbm.at[pl.ds(w * BPW, BPW)])


def _sc_pool(arr, p_lin):
    mesh = plsc.VectorSubcoreMesh(core_axis_name="c", subcore_axis_name="s")
    return pl.kernel(
        _sc_body,
        out_type=jax.ShapeDtypeStruct((B, PD), jnp.float32),
        mesh=mesh,
        scratch_types=[
            pltpu.VMEM((GPW, IPG), jnp.int32),
            pltpu.VMEM((2, IPG, PD), jnp.float32),
            pltpu.VMEM((BPW, PD), jnp.float32),
            pltpu.SemaphoreType.DMA,
            pltpu.SemaphoreType.DMA,
        ],
        compiler_params=pltpu.CompilerParams(use_tc_tiling_on_sc=False),
    )(arr, p_lin)


def _tail_body(s_ref, b_ref, o_ref):
    logits = s_ref[...][:, :EN] + b_ref[...]
    m = jnp.max(logits, axis=1, keepdims=True)
    sh = logits - m
    o_ref[...] = sh - jnp.log(jnp.sum(jnp.exp(sh), axis=1, keepdims=True))


def _tail(sums, b):
    return pl.pallas_call(
        _tail_body,
        out_shape=jax.ShapeDtypeStruct((B, EN), jnp.float32),
    )(sums, b.reshape(1, EN))


def kernel(text, table, W, b):
    W16s = jnp.zeros((PD, D), jnp.float32).at[:EN].set(W) * (1.0 / L)
    pt = _project(table.T, W16s)
    p_lin = pt.T
    arr = text.astype(jnp.int32).reshape(NW, GPW, IPG)
    sums = _sc_pool(arr, p_lin)
    return _tail(sums, b)


# speedup vs baseline: 3.4945x; 3.2069x over previous
import jax
import jax.numpy as jnp
from jax import lax
from jax.experimental import pallas as pl
from jax.experimental.pallas import tpu as pltpu
from jax.experimental.pallas import tpu_sc as plsc

NC, NS = 2, 16
NW = NC * NS
B, L, D = 4096, 200, 64
V = 1000000
EN = 5
PD = 16
IPG = 100
GPW = 2 * (B // NW)
BPW = B // NW
NG = 128 // PD
VG = V // NG
PC = 2048
DG = [(VG * g) % PC for g in range(NG)]
OG = [VG * g - DG[g] for g in range(NG)]
NB = -(-(VG + max(DG)) // PC)
VP = NB * PC


def _project(tableT, W16s):
    in_specs = [pl.BlockSpec((PD, D), lambda i: (0, 0))] + [
        pl.BlockSpec((D, PC), lambda i, g=g: (0, OG[g] // PC + i))
        for g in range(NG)
    ]

    def body(w_ref, *refs):
        t_refs, o_ref = refs[:-1], refs[-1]
        xs = [
            lax.dot_general(
                w_ref[...], t_refs[g][...],
                (((1,), (0,)), ((), ())),
                preferred_element_type=jnp.float32,
            )
            for g in range(NG)
        ]
        o_ref[...] = lax.transpose(jnp.concatenate(xs, axis=0), (1, 0))

    return pl.pallas_call(
        body,
        grid=(NB,),
        in_specs=in_specs,
        out_specs=pl.BlockSpec((PC, 128), lambda i: (i, 0)),
        out_shape=jax.ShapeDtypeStruct((VP, 128), jnp.float32),
        compiler_params=pltpu.CompilerParams(fuse_transposed_lhs_in_matmul=True),
    )(W16s, *([tableT] * NG))


def _sc_body(idx_hbm, p_hbm, out_hbm, idx_v, rows_v, out_v, sem0, sem1):
    w = lax.axis_index("s") * NC + lax.axis_index("c")
    pltpu.sync_copy(idx_hbm.at[w], idx_v)
    sems = (sem0, sem1)

    pltpu.async_copy(p_hbm.at[idx_v.at[0]], rows_v.at[0], sem0)

    def row_body(r, _):
        acc = jnp.zeros((PD,), jnp.float32)
        for h in range(2):
            g = r * 2 + h
            nxt = jnp.minimum(g + 1, GPW - 1)
            pltpu.async_copy(
                p_hbm.at[idx_v.at[nxt]], rows_v.at[1 - h], sems[1 - h]
            )
            pltpu.make_async_copy(
                p_hbm.at[idx_v.at[g]], rows_v.at[h], sems[h]
            ).wait()

            def kbody(k, a):
                r0 = rows_v[h, k * 4 + 0, :]
                r1 = rows_v[h, k * 4 + 1, :]
                r2 = rows_v[h, k * 4 + 2, :]
                r3 = rows_v[h, k * 4 + 3, :]
                return a + ((r0 + r1) + (r2 + r3))

            acc = lax.fori_loop(0, IPG // 4, kbody, acc)
        out_v[r, :] = acc
        return 0

    lax.fori_loop(0, BPW, row_body, 0)
    pltpu.make_async_copy(
        p_hbm.at[idx_v.at[GPW - 1]], rows_v.at[0], sem0
    ).wait()
    pltpu.sync_copy(out_v, out_hbm.at[pl.ds(w * BPW, BPW)])


def _sc_pool(arr, p_lin):
    mesh = plsc.VectorSubcoreMesh(core_axis_name="c", subcore_axis_name="s")
    return pl.kernel(
        _sc_body,
        out_type=jax.ShapeDtypeStruct((B, PD), jnp.float32),
        mesh=mesh,
        scratch_types=[
            pltpu.VMEM((GPW, IPG), jnp.int32),
            pltpu.VMEM((2, IPG, PD), jnp.float32),
            pltpu.VMEM((BPW, PD), jnp.float32),
            pltpu.SemaphoreType.DMA,
            pltpu.SemaphoreType.DMA,
        ],
        compiler_params=pltpu.CompilerParams(use_tc_tiling_on_sc=False),
    )(arr, p_lin)


def _tail_body(s_ref, b_ref, o_ref):
    logits = s_ref[...][:, :EN] + b_ref[...]
    m = jnp.max(logits, axis=1, keepdims=True)
    sh = logits - m
    o_ref[...] = sh - jnp.log(jnp.sum(jnp.exp(sh), axis=1, keepdims=True))


def _tail(sums, b):
    return pl.pallas_call(
        _tail_body,
        out_shape=jax.ShapeDtypeStruct((B, EN), jnp.float32),
    )(sums, b.reshape(1, EN))


def kernel(text, table, W, b):
    W16s = jnp.zeros((PD, D), jnp.float32).at[:EN].set(W) * (1.0 / L)
    p_pack = _project(table.T, W16s)
    p_lin = p_pack.reshape(NG * VP, PD)
    t32 = text.astype(jnp.int32)
    g32 = t32 // VG
    arr = (NG * (t32 % VG + (VG * g32) % PC) + g32).reshape(NW, GPW, IPG)
    sums = _sc_pool(arr, p_lin)
    return _tail(sums, b)


# speedup vs baseline: 4.3123x; 1.2340x over previous
import jax
import jax.numpy as jnp
from jax import lax
from jax.experimental import pallas as pl
from jax.experimental.pallas import tpu as pltpu
from jax.experimental.pallas import tpu_sc as plsc

NC, NS = 2, 16
NW = NC * NS
B, L, D = 4096, 200, 64
V = 1000000
EN = 5
PD = 16
IPG = 100
GPW = 2 * (B // NW)
BPW = B // NW
NG = 128 // PD
VG = V // NG
PC = 4096
DG = [(VG * g) % PC for g in range(NG)]
OG = [VG * g - DG[g] for g in range(NG)]
NB = -(-(VG + max(DG)) // PC)
VP = NB * PC


def _project(tableT, W16s):
    in_specs = [pl.BlockSpec((PD, D), lambda i: (0, 0))] + [
        pl.BlockSpec((D, PC), lambda i, g=g: (0, OG[g] // PC + i))
        for g in range(NG)
    ]

    def body(w_ref, *refs):
        t_refs, o_ref = refs[:-1], refs[-1]
        xs = [
            lax.dot_general(
                w_ref[...], t_refs[g][...],
                (((1,), (0,)), ((), ())),
                preferred_element_type=jnp.float32,
            )
            for g in range(NG)
        ]
        o_ref[...] = lax.transpose(jnp.concatenate(xs, axis=0), (1, 0))

    return pl.pallas_call(
        body,
        grid=(NB,),
        in_specs=in_specs,
        out_specs=pl.BlockSpec((PC, 128), lambda i: (i, 0)),
        out_shape=jax.ShapeDtypeStruct((VP, 128), jnp.float32),
        compiler_params=pltpu.CompilerParams(fuse_transposed_lhs_in_matmul=True),
    )(W16s, *([tableT] * NG))


def _gsum(rows_v, j):
    vals = [rows_v[j, k, :] for k in range(IPG)]
    while len(vals) > 1:
        vals = [
            vals[i] + vals[i + 1] if i + 1 < len(vals) else vals[i]
            for i in range(0, len(vals), 2)
        ]
    return vals[0]


def _sc_body(idx_hbm, p_hbm, out_hbm, idx_v, rows_v, out_v, *sems):
    w = lax.axis_index("s") * NC + lax.axis_index("c")
    pltpu.sync_copy(idx_hbm.at[w], idx_v)

    for j in range(3):
        pltpu.async_copy(p_hbm.at[idx_v.at[j]], rows_v.at[j], sems[j])

    def row_pair(rr, _):
        g0 = rr * 4
        accs = []
        for j in range(4):
            nxt = jnp.minimum(g0 + j + 3, GPW - 1)
            pltpu.async_copy(
                p_hbm.at[idx_v.at[nxt]], rows_v.at[(j + 3) % 4], sems[(j + 3) % 4]
            )
            pltpu.make_async_copy(
                p_hbm.at[idx_v.at[g0 + j]], rows_v.at[j], sems[j]
            ).wait()
            accs.append(_gsum(rows_v, j))
        out_v[rr * 2, :] = accs[0] + accs[1]
        out_v[rr * 2 + 1, :] = accs[2] + accs[3]
        return 0

    lax.fori_loop(0, BPW // 2, row_pair, 0)
    for j in (0, 1, 2):
        pltpu.make_async_copy(
            p_hbm.at[idx_v.at[GPW - 1]], rows_v.at[j], sems[j]
        ).wait()
    pltpu.sync_copy(out_v, out_hbm.at[pl.ds(w * BPW, BPW)])


def _sc_pool(arr, p_lin):
    mesh = plsc.VectorSubcoreMesh(core_axis_name="c", subcore_axis_name="s")
    return pl.kernel(
        _sc_body,
        out_type=jax.ShapeDtypeStruct((B, PD), jnp.float32),
        mesh=mesh,
        scratch_types=[
            pltpu.VMEM((GPW, IPG), jnp.int32),
            pltpu.VMEM((4, IPG, PD), jnp.float32),
            pltpu.VMEM((BPW, PD), jnp.float32),
            pltpu.SemaphoreType.DMA,
            pltpu.SemaphoreType.DMA,
            pltpu.SemaphoreType.DMA,
            pltpu.SemaphoreType.DMA,
        ],
        compiler_params=pltpu.CompilerParams(use_tc_tiling_on_sc=False),
    )(arr, p_lin)


def _tail_body(s_ref, b_ref, o_ref):
    logits = s_ref[...][:, :EN] + b_ref[...]
    m = jnp.max(logits, axis=1, keepdims=True)
    sh = logits - m
    o_ref[...] = sh - jnp.log(jnp.sum(jnp.exp(sh), axis=1, keepdims=True))


def _tail(sums, b):
    return pl.pallas_call(
        _tail_body,
        out_shape=jax.ShapeDtypeStruct((B, EN), jnp.float32),
    )(sums, b.reshape(1, EN))


def kernel(text, table, W, b):
    W16s = jnp.zeros((PD, D), jnp.float32).at[:EN].set(W) * (1.0 / L)
    p_pack = _project(table.T, W16s)
    p_lin = p_pack.reshape(NG * VP, PD)
    t32 = text.astype(jnp.int32)
    g32 = t32 // VG
    arr = (NG * (t32 % VG + (VG * g32) % PC) + g32).reshape(NW, GPW, IPG)
    sums = _sc_pool(arr, p_lin)
    return _tail(sums, b)


# speedup vs baseline: 4.3711x; 1.0136x over previous
import jax
import jax.numpy as jnp
from jax import lax
from jax.experimental import pallas as pl
from jax.experimental.pallas import tpu as pltpu
from jax.experimental.pallas import tpu_sc as plsc

NC, NS = 2, 16
NW = NC * NS
B, L, D = 4096, 200, 64
V = 1000000
EN = 5
PD = 8
IPG = 100
GPW = 2 * (B // NW)
BPW = B // NW
NG = 128 // PD
VG = V // NG
PC = 4096
DG = [(VG * g) % PC for g in range(NG)]
OG = [VG * g - DG[g] for g in range(NG)]
NB = -(-(VG + max(DG)) // PC)
VP = NB * PC


def _project(tableT, W16s):
    in_specs = [pl.BlockSpec((PD, D), lambda i: (0, 0))] + [
        pl.BlockSpec((D, PC), lambda i, g=g: (0, OG[g] // PC + i))
        for g in range(NG)
    ]

    def body(w_ref, *refs):
        t_refs, o_ref = refs[:-1], refs[-1]
        xs = [
            lax.dot_general(
                w_ref[...], t_refs[g][...],
                (((1,), (0,)), ((), ())),
                preferred_element_type=jnp.float32,
            )
            for g in range(NG)
        ]
        o_ref[...] = lax.transpose(jnp.concatenate(xs, axis=0), (1, 0))

    return pl.pallas_call(
        body,
        grid=(NB,),
        in_specs=in_specs,
        out_specs=pl.BlockSpec((PC, 128), lambda i: (i, 0)),
        out_shape=jax.ShapeDtypeStruct((VP, 128), jnp.float32),
        compiler_params=pltpu.CompilerParams(fuse_transposed_lhs_in_matmul=True),
    )(W16s, *([tableT] * NG))


def _gsum(rows_v, j, h2, c8):
    rv = rows_v.at[j]
    vals = [
        plsc.load_gather(rv, [h2 + 2 * k, c8]) for k in range(IPG // 2)
    ]
    while len(vals) > 1:
        vals = [
            vals[i] + vals[i + 1] if i + 1 < len(vals) else vals[i]
            for i in range(0, len(vals), 2)
        ]
    return vals[0]


def _sc_body(idx_hbm, p_hbm, out_hbm, idx_v, rows_v, out_v, *sems):
    w = lax.axis_index("s") * NC + lax.axis_index("c")
    pltpu.sync_copy(idx_hbm.at[w], idx_v)
    i16 = lax.iota(jnp.int32, 16)
    h2 = i16 >> 3
    c8 = i16 & 7

    for j in range(3):
        pltpu.async_copy(p_hbm.at[idx_v.at[j]], rows_v.at[j], sems[j])

    def row_pair(rr, _):
        g0 = rr * 4
        accs = []
        for j in range(4):
            nxt = jnp.minimum(g0 + j + 3, GPW - 1)
            pltpu.async_copy(
                p_hbm.at[idx_v.at[nxt]], rows_v.at[(j + 3) % 4], sems[(j + 3) % 4]
            )
            pltpu.make_async_copy(
                p_hbm.at[idx_v.at[g0 + j]], rows_v.at[j], sems[j]
            ).wait()
            accs.append(_gsum(rows_v, j, h2, c8))
        out_v[rr * 2, :] = accs[0] + accs[1]
        out_v[rr * 2 + 1, :] = accs[2] + accs[3]
        return 0

    lax.fori_loop(0, BPW // 2, row_pair, 0)
    for j in (0, 1, 2):
        pltpu.make_async_copy(
            p_hbm.at[idx_v.at[GPW - 1]], rows_v.at[j], sems[j]
        ).wait()
    pltpu.sync_copy(out_v, out_hbm.at[pl.ds(w * BPW, BPW)])


def _sc_pool(arr, p_lin):
    mesh = plsc.VectorSubcoreMesh(core_axis_name="c", subcore_axis_name="s")
    return pl.kernel(
        _sc_body,
        out_type=jax.ShapeDtypeStruct((B, 16), jnp.float32),
        mesh=mesh,
        scratch_types=[
            pltpu.VMEM((GPW, IPG), jnp.int32),
            pltpu.VMEM((4, IPG, PD), jnp.float32),
            pltpu.VMEM((BPW, 16), jnp.float32),
            pltpu.SemaphoreType.DMA,
            pltpu.SemaphoreType.DMA,
            pltpu.SemaphoreType.DMA,
            pltpu.SemaphoreType.DMA,
        ],
        compiler_params=pltpu.CompilerParams(
            use_tc_tiling_on_sc=False, needs_layout_passes=False
        ),
    )(arr, p_lin)


def _tail_body(s_ref, b_ref, o_ref):
    s = s_ref[...]
    logits = s[:, :EN] + s[:, PD:PD + EN] + b_ref[...]
    m = jnp.max(logits, axis=1, keepdims=True)
    sh = logits - m
    o_ref[...] = sh - jnp.log(jnp.sum(jnp.exp(sh), axis=1, keepdims=True))


def _tail(sums, b):
    return pl.pallas_call(
        _tail_body,
        out_shape=jax.ShapeDtypeStruct((B, EN), jnp.float32),
    )(sums, b.reshape(1, EN))


def kernel(text, table, W, b):
    W16s = jnp.zeros((PD, D), jnp.float32).at[:EN].set(W) * (1.0 / L)
    p_pack = _project(table.T, W16s)
    p_lin = p_pack.reshape(NG * VP, PD)
    t32 = text.astype(jnp.int32)
    g32 = t32 // VG
    arr = (NG * (t32 % VG + (VG * g32) % PC) + g32).reshape(NW, GPW, IPG)
    sums = _sc_pool(arr, p_lin)
    return _tail(sums, b)


# speedup vs baseline: 4.4278x; 1.0130x over previous
import jax
import jax.numpy as jnp
from jax import lax
from jax.experimental import pallas as pl
from jax.experimental.pallas import tpu as pltpu
from jax.experimental.pallas import tpu_sc as plsc

NC, NS = 2, 16
NW = NC * NS
B, L, D = 4096, 200, 64
V = 1000000
EN = 5
PD = 8
IPG = 200
GPW = B // NW
BPW = B // NW
NG = 128 // PD
VG = V // NG
PC = 4096
DG = [(VG * g) % PC for g in range(NG)]
OG = [VG * g - DG[g] for g in range(NG)]
NB = -(-(VG + max(DG)) // PC)
VP = NB * PC


def _project(tableT, W16s):
    in_specs = [pl.BlockSpec((PD, D), lambda i: (0, 0))] + [
        pl.BlockSpec((D, PC), lambda i, g=g: (0, OG[g] // PC + i))
        for g in range(NG)
    ]

    def body(w_ref, *refs):
        t_refs, o_ref = refs[:-1], refs[-1]
        xs = [
            lax.dot_general(
                w_ref[...], t_refs[g][...],
                (((1,), (0,)), ((), ())),
                preferred_element_type=jnp.float32,
            )
            for g in range(NG)
        ]
        o_ref[...] = lax.transpose(jnp.concatenate(xs, axis=0), (1, 0))

    return pl.pallas_call(
        body,
        grid=(NB,),
        in_specs=in_specs,
        out_specs=pl.BlockSpec((PC, 128), lambda i: (i, 0)),
        out_shape=jax.ShapeDtypeStruct((VP, 128), jnp.float32),
        compiler_params=pltpu.CompilerParams(fuse_transposed_lhs_in_matmul=True),
    )(W16s, *([tableT] * NG))


def _gsum(rows_v, j, h2, c8):
    rv = rows_v.at[j]
    vals = [
        plsc.load_gather(rv, [h2 + 2 * k, c8]) for k in range(IPG // 2)
    ]
    while len(vals) > 1:
        vals = [
            vals[i] + vals[i + 1] if i + 1 < len(vals) else vals[i]
            for i in range(0, len(vals), 2)
        ]
    return vals[0]


def _sc_body(idx_hbm, p_hbm, out_hbm, idx_v, rows_v, out_v, *sems):
    w = lax.axis_index("s") * NC + lax.axis_index("c")
    pltpu.sync_copy(idx_hbm.at[w], idx_v)
    i16 = lax.iota(jnp.int32, 16)
    h2 = i16 >> 3
    c8 = i16 & 7

    for j in range(3):
        pltpu.async_copy(p_hbm.at[idx_v.at[j]], rows_v.at[j], sems[j])

    def row_quad(rr, _):
        g0 = rr * 4
        for j in range(4):
            nxt = jnp.minimum(g0 + j + 3, GPW - 1)
            pltpu.async_copy(
                p_hbm.at[idx_v.at[nxt]], rows_v.at[(j + 3) % 4], sems[(j + 3) % 4]
            )
            pltpu.make_async_copy(
                p_hbm.at[idx_v.at[g0 + j]], rows_v.at[j], sems[j]
            ).wait()
            out_v[g0 + j, :] = _gsum(rows_v, j, h2, c8)
        return 0

    lax.fori_loop(0, BPW // 4, row_quad, 0)
    for j in (0, 1, 2):
        pltpu.make_async_copy(
            p_hbm.at[idx_v.at[GPW - 1]], rows_v.at[j], sems[j]
        ).wait()
    pltpu.sync_copy(out_v, out_hbm.at[pl.ds(w * BPW, BPW)])


def _sc_pool(arr, p_lin):
    mesh = plsc.VectorSubcoreMesh(core_axis_name="c", subcore_axis_name="s")
    return pl.kernel(
        _sc_body,
        out_type=jax.ShapeDtypeStruct((B, 16), jnp.float32),
        mesh=mesh,
        scratch_types=[
            pltpu.VMEM((GPW, IPG), jnp.int32),
            pltpu.VMEM((4, IPG, PD), jnp.float32),
            pltpu.VMEM((BPW, 16), jnp.float32),
            pltpu.SemaphoreType.DMA,
            pltpu.SemaphoreType.DMA,
            pltpu.SemaphoreType.DMA,
            pltpu.SemaphoreType.DMA,
        ],
        compiler_params=pltpu.CompilerParams(
            use_tc_tiling_on_sc=False, needs_layout_passes=False
        ),
    )(arr, p_lin)


def _tail_body(s_ref, b_ref, o_ref):
    s = s_ref[...]
    logits = s[:, :EN] + s[:, PD:PD + EN] + b_ref[...]
    m = jnp.max(logits, axis=1, keepdims=True)
    sh = logits - m
    o_ref[...] = sh - jnp.log(jnp.sum(jnp.exp(sh), axis=1, keepdims=True))


def _tail(sums, b):
    return pl.pallas_call(
        _tail_body,
        out_shape=jax.ShapeDtypeStruct((B, EN), jnp.float32),
    )(sums, b.reshape(1, EN))


def kernel(text, table, W, b):
    W16s = jnp.zeros((PD, D), jnp.float32).at[:EN].set(W) * (1.0 / L)
    p_pack = _project(table.T, W16s)
    p_lin = p_pack.reshape(NG * VP, PD)
    t32 = text.astype(jnp.int32)
    g32 = t32 // VG
    arr = (NG * (t32 % VG + (VG * g32) % PC) + g32).reshape(NW, GPW, IPG)
    sums = _sc_pool(arr, p_lin)
    return _tail(sums, b)
